# baseline (speedup 1.0000x reference)
import functools

import jax
import jax.numpy as jnp
from jax import lax
from jax.experimental import pallas as pl
from jax.experimental.pallas import tpu as pltpu
from jax.experimental.pallas import tpu_sc as plsc

_NC = 2
_NS = 16
_NW = _NC * _NS
_L = 16
_C = 128


@functools.lru_cache(maxsize=None)
def _build(N, D, n_chunks):
    n_pairs = n_chunks // 2
    mesh = plsc.VectorSubcoreMesh(core_axis_name="c", subcore_axis_name="s")

    @functools.partial(
        pl.kernel,
        mesh=mesh,
        compiler_params=pltpu.CompilerParams(use_tc_tiling_on_sc=False),
        out_type=jax.ShapeDtypeStruct((_NW, _L), jnp.float32),
        scratch_types=[
            pltpu.VMEM((n_chunks, _C), jnp.int32),
            pltpu.VMEM((_C, D), jnp.float32),
            pltpu.VMEM((_C, D), jnp.float32),
            pltpu.VMEM((_C, D), jnp.float32),
            pltpu.VMEM((_C, D), jnp.float32),
            pltpu.VMEM((_L,), jnp.float32),
            pltpu.SemaphoreType.DMA,
            pltpu.SemaphoreType.DMA,
            pltpu.SemaphoreType.DMA,
            pltpu.SemaphoreType.DMA,
        ],
    )
    def k(preds_hbm, idx_hbm, table_hbm, out_hbm,
          idx_all, p_a, p_b, r_a, r_b, acc_st, sp_a, sp_b, sr_a, sr_b):
        wid = lax.axis_index("s") * _NC + lax.axis_index("c")
        p_bufs = (p_a, p_b)
        r_bufs = (r_a, r_b)
        sp = (sp_a, sp_b)
        sr = (sr_a, sr_b)

        pltpu.sync_copy(idx_hbm.at[wid], idx_all)
        w_row0 = wid * (n_chunks * _C)

        def issue(c, b):
            rb = w_row0 + c * _C
            pltpu.async_copy(preds_hbm.at[pl.ds(rb, _C)], p_bufs[b], sp[b])
            pltpu.async_copy(table_hbm.at[idx_all.at[c]], r_bufs[b], sr[b])

        def wait(c, b):
            rb = w_row0 + c * _C
            pltpu.make_async_copy(
                preds_hbm.at[pl.ds(rb, _C)], p_bufs[b], sp[b]).wait()
            pltpu.make_async_copy(
                table_hbm.at[idx_all.at[c]], r_bufs[b], sr[b]).wait()

        def chunk_sum(b, accs):
            pv = p_bufs[b]
            rv = r_bufs[b]

            def row_body(r, accs):
                a = list(accs)
                for rr in range(2):
                    row = r * 2 + rr
                    for d in range(D // _L):
                        dp = (pv[row, pl.ds(d * _L, _L)]
                              - rv[row, pl.ds(d * _L, _L)])
                        a[d] = a[d] + dp * dp
                return tuple(a)

            return lax.fori_loop(0, _C // 2, row_body, accs)

        issue(0, 0)
        zero = jnp.zeros((_L,), jnp.float32)

        def pair_body(g, accs):
            issue(2 * g + 1, 1)
            wait(2 * g, 0)
            accs = chunk_sum(0, accs)

            @pl.when(g < n_pairs - 1)
            def _():
                issue(2 * g + 2, 0)

            wait(2 * g + 1, 1)
            accs = chunk_sum(1, accs)
            return accs

        accs = lax.fori_loop(0, n_pairs, pair_body, (zero, zero, zero, zero))
        acc_st[...] = (accs[0] + accs[1]) + (accs[2] + accs[3])
        pltpu.sync_copy(acc_st, out_hbm.at[wid])

    return k


def kernel(preds, target, table):
    B, S, D = preds.shape
    N = B * S
    per_w = N // _NW
    n_chunks = per_w // _C
    k = _build(N, D, n_chunks)
    partials = k(
        preds.reshape(N, D),
        target.reshape(_NW, n_chunks, _C),
        table,
    )
    return jnp.sum(partials) * jnp.float32(1.0 / (N * D))


# speedup vs baseline: 1.0599x; 1.0599x over previous
import functools

import jax
import jax.numpy as jnp
from jax import lax
from jax.experimental import pallas as pl
from jax.experimental.pallas import tpu as pltpu
from jax.experimental.pallas import tpu_sc as plsc

_NC = 2
_NS = 16
_NW = _NC * _NS
_L = 16
_C = 128


@functools.lru_cache(maxsize=None)
def _build(N, D, V2, n_chunks):
    n_pairs = n_chunks // 2
    mesh = plsc.VectorSubcoreMesh(core_axis_name="c", subcore_axis_name="s")

    @functools.partial(
        pl.kernel,
        mesh=mesh,
        out_type=jax.ShapeDtypeStruct((_NW, _L), jnp.float32),
        scratch_types=[
            pltpu.VMEM((n_chunks, _C), jnp.int32),
            pltpu.VMEM((n_chunks, _C), jnp.int32),
            pltpu.VMEM((_C, D), jnp.float32),
            pltpu.VMEM((_C, D), jnp.float32),
            pltpu.VMEM((_C, 2 * D), jnp.float32),
            pltpu.VMEM((_C, 2 * D), jnp.float32),
            pltpu.VMEM((_L,), jnp.float32),
            pltpu.SemaphoreType.DMA,
            pltpu.SemaphoreType.DMA,
            pltpu.SemaphoreType.DMA,
            pltpu.SemaphoreType.DMA,
        ],
    )
    def k(preds_hbm, vidx_hbm, poff_hbm, table_hbm, out_hbm,
          idx_all, off_all, p_a, p_b, r_a, r_b, acc_st,
          sp_a, sp_b, sr_a, sr_b):
        wid = lax.axis_index("s") * _NC + lax.axis_index("c")
        p_bufs = (p_a, p_b)
        r_bufs = (r_a, r_b)
        sp = (sp_a, sp_b)
        sr = (sr_a, sr_b)

        pltpu.sync_copy(vidx_hbm.at[wid], idx_all)
        pltpu.sync_copy(poff_hbm.at[wid], off_all)
        w_row0 = wid * (n_chunks * _C)

        def issue(c, b):
            rb = w_row0 + c * _C
            pltpu.async_copy(preds_hbm.at[pl.ds(rb, _C)], p_bufs[b], sp[b])
            pltpu.async_copy(table_hbm.at[idx_all.at[c]], r_bufs[b], sr[b])

        def wait(c, b):
            rb = w_row0 + c * _C
            pltpu.make_async_copy(
                preds_hbm.at[pl.ds(rb, _C)], p_bufs[b], sp[b]).wait()
            pltpu.make_async_copy(
                table_hbm.at[idx_all.at[c]], r_bufs[b], sr[b]).wait()

        def chunk_sum(c, b, accs):
            pv = p_bufs[b]
            rv = r_bufs[b]

            def group_body(q, accs):
                a = list(accs)
                off_v = off_all[c, pl.ds(q * _L, _L)]
                for j in range(_L):
                    row = q * _L + j
                    off = off_v[j]
                    for d in range(D // _L):
                        dp = (pv[row, pl.ds(d * _L, _L)]
                              - rv[row, pl.ds(off + d * _L, _L)])
                        a[d] = a[d] + dp * dp
                return tuple(a)

            return lax.fori_loop(0, _C // _L, group_body, accs)

        issue(0, 0)
        zero = jnp.zeros((_L,), jnp.float32)

        def pair_body(g, accs):
            issue(2 * g + 1, 1)
            wait(2 * g, 0)
            accs = chunk_sum(2 * g, 0, accs)

            @pl.when(g < n_pairs - 1)
            def _():
                issue(2 * g + 2, 0)

            wait(2 * g + 1, 1)
            accs = chunk_sum(2 * g + 1, 1, accs)
            return accs

        accs = lax.fori_loop(0, n_pairs, pair_body, (zero, zero, zero, zero))
        acc_st[...] = (accs[0] + accs[1]) + (accs[2] + accs[3])
        pltpu.sync_copy(acc_st, out_hbm.at[wid])

    return k


def kernel(preds, target, table):
    B, S, D = preds.shape
    V = table.shape[0]
    N = B * S
    per_w = N // _NW
    n_chunks = per_w // _C
    k = _build(N, D, V // 2, n_chunks)
    tgt = target.reshape(_NW, n_chunks, _C)
    partials = k(
        preds.reshape(N, D),
        tgt >> 1,
        (tgt & 1) * D,
        table.reshape(V // 2, 2 * D),
    )
    return jnp.sum(partials) * jnp.float32(1.0 / (N * D))
